# baseline (speedup 1.0000x reference)
import functools
import math

import jax
import jax.numpy as jnp
from jax import lax
from jax.experimental import pallas as pl
from jax.experimental.pallas import tpu as pltpu
from jax.experimental.pallas import tpu_sc as plsc

_B = 32768
_ND = 64
_L = 16
_ROWS = _B // _L
_TROWS = _ROWS // 16
_EXP_NEG5 = math.exp(-5.0)
_LN2 = 0.6931471805599453


def _softlog(x):
    bits = plsc.bitcast(x, jnp.int32)
    ex = (bits >> 23) & 0xFF
    m = plsc.bitcast((bits & 0x7FFFFF) | 0x3F800000, jnp.float32)
    big = m > 1.4142135381698608
    m2 = jnp.where(big, m * 0.5, m)
    ef = (ex - jnp.where(big, 126, 127)).astype(jnp.float32)
    r = (m2 - 1.0) / (m2 + 1.0)
    r2 = r * r
    poly = 1.0 + r2 * (1.0 / 3.0 + r2 * (1.0 / 5.0 + r2 * (1.0 / 7.0 + r2 * (1.0 / 9.0))))
    return ef * _LN2 + 2.0 * r * poly


def _body(sc_h, lab_h, dat_h,
          out_h, parts_h, fin_h,
          sc_v, lab_v, dat_v, p_v,
          acc, part_r, parts_v, t_tab, lz_tab, outv_r):
    sid = lax.axis_index("s")
    iota = lax.broadcasted_iota(jnp.int32, (_L,), 0)
    lane64 = iota * _ND

    r0 = sid * _TROWS
    pltpu.sync_copy(sc_h.at[pl.ds(r0, _TROWS)], sc_v)
    pltpu.sync_copy(lab_h.at[pl.ds(r0, _TROWS)], lab_v)
    pltpu.sync_copy(dat_h.at[pl.ds(r0, _TROWS)], dat_v)

    def zero_body(j, c):
        acc[pl.ds(j * _L, _L)] = jnp.zeros((_L,), jnp.float32)
        return c
    lax.fori_loop(0, (32 * _ND) // _L, zero_body, 0)

    iv2 = 2 * iota

    def p1_body(i, c):
        d = dat_v[i]
        idx = lane64 + ((d + iota) & (_ND - 1))
        irow = jnp.full((_L,), 0, jnp.int32) + i
        s0 = plsc.load_gather(sc_v, [irow, iv2])
        s1 = plsc.load_gather(sc_v, [irow, iv2 + 1])
        x = s1 - s0
        p = 1.0 / (1.0 + jnp.exp(-x))
        p_v[i] = p
        plsc.addupdate_scatter(acc, [idx], ((lab_v[i] << 12) + 1).astype(jnp.float32))
        plsc.addupdate_scatter(acc, [idx + 16 * _ND], jnp.exp(p))
        return c
    lax.fori_loop(0, _TROWS, p1_body, 0)
    plsc.subcore_barrier()

    for v in range(4):
        dv = v * _L + iota
        t_cn = jnp.zeros((_L,), jnp.float32)
        t_e = jnp.zeros((_L,), jnp.float32)
        for j in range(16):
            idx = j * _ND + ((dv + j) & (_ND - 1))
            t_cn = t_cn + plsc.load_gather(acc, [idx])
            t_e = t_e + plsc.load_gather(acc, [idx + 16 * _ND])
        icn = t_cn.astype(jnp.int32)
        part_r[pl.ds(v * _L, _L)] = (icn & 0xFFF).astype(jnp.float32)
        part_r[pl.ds(_ND + v * _L, _L)] = (icn >> 12).astype(jnp.float32)
        part_r[pl.ds(2 * _ND + v * _L, _L)] = t_e
    pltpu.sync_copy(part_r, parts_h.at[sid])
    plsc.subcore_barrier()
    pltpu.sync_copy(parts_h, parts_v)

    nvalid = jnp.float32(0.0)
    for v in range(4):
        cnt = parts_v[0, pl.ds(v * _L, _L)]
        n1 = parts_v[0, pl.ds(_ND + v * _L, _L)]
        e_tot = parts_v[0, pl.ds(2 * _ND + v * _L, _L)]
        for s in range(1, 16):
            cnt = cnt + parts_v[s, pl.ds(v * _L, _L)]
            n1 = n1 + parts_v[s, pl.ds(_ND + v * _L, _L)]
            e_tot = e_tot + parts_v[s, pl.ds(2 * _ND + v * _L, _L)]
        valid = cnt >= 2.0
        has1 = n1 > 0.5
        s_den = jnp.where(has1, n1 + (cnt - n1) * _EXP_NEG5, cnt)
        s_den = jnp.maximum(s_den, 1e-30)
        va = jnp.where(valid, jnp.where(has1, _EXP_NEG5, 1.0) / s_den, 0.0)
        vb = jnp.where(valid, 1.0 / s_den, 0.0)
        log_e = _softlog(jnp.maximum(e_tot, 1e-30))
        dv = v * _L + iota
        plsc.store_scatter(t_tab, [2 * dv], va)
        plsc.store_scatter(t_tab, [2 * dv + 1], vb)
        lz_tab[pl.ds(v * _L, _L)] = log_e
        nvalid = nvalid + jnp.sum(jnp.where(valid, 1.0, 0.0))

    def p4_body(i, acc_c):
        d = dat_v[i]
        t = plsc.load_gather(t_tab, [2 * d + lab_v[i]])
        log_e = plsc.load_gather(lz_tab, [d])
        return acc_c + t * (log_e - p_v[i])
    accv = lax.fori_loop(0, _TROWS, p4_body, jnp.zeros((_L,), jnp.float32))

    part = jnp.sum(accv)
    part_r[pl.ds(0, _L)] = jnp.where(iota == 0, part, 0.0)
    pltpu.sync_copy(part_r, fin_h.at[sid])
    plsc.subcore_barrier()

    @pl.when(sid == 0)
    def _():
        pltpu.sync_copy(fin_h, parts_v)
        tot = parts_v[0, pl.ds(0, _L)]
        for s in range(1, 16):
            tot = tot + parts_v[s, pl.ds(0, _L)]
        denv = jnp.maximum(nvalid, 1.0) + jnp.zeros((_L,), jnp.float32)
        outv_r[...] = jnp.where(iota == 0, tot, 0.0) / denv
        pltpu.sync_copy(outv_r, out_h)


_sc_loss = functools.partial(
    pl.kernel,
    out_type=(
        jax.ShapeDtypeStruct((_L,), jnp.float32),
        jax.ShapeDtypeStruct((16, 3 * _ND), jnp.float32),
        jax.ShapeDtypeStruct((16, 3 * _ND), jnp.float32),
    ),
    mesh=plsc.VectorSubcoreMesh(core_axis_name="c", subcore_axis_name="s",
                                num_cores=1),
    compiler_params=pltpu.CompilerParams(needs_layout_passes=False),
    scratch_types=[
        pltpu.VMEM((_TROWS, 2 * _L), jnp.float32),
        pltpu.VMEM((_TROWS, _L), jnp.int32),
        pltpu.VMEM((_TROWS, _L), jnp.int32),
        pltpu.VMEM((_TROWS, _L), jnp.float32),
        pltpu.VMEM((32 * _ND,), jnp.float32),
        pltpu.VMEM((3 * _ND,), jnp.float32),
        pltpu.VMEM((16, 3 * _ND), jnp.float32),
        pltpu.VMEM((2 * _ND,), jnp.float32),
        pltpu.VMEM((_ND,), jnp.float32),
        pltpu.VMEM((_L,), jnp.float32),
    ],
)(_body)


def kernel(scores, labels, dates):
    sc = scores.reshape(_ROWS, 2 * _L)
    lab = labels.astype(jnp.int32).reshape(_ROWS, _L)
    dat = dates.astype(jnp.int32).reshape(_ROWS, _L)
    out, _, _ = _sc_loss(sc, lab, dat)
    return out[0]


# speedup vs baseline: 1.4555x; 1.4555x over previous
import functools
import math

import jax
import jax.numpy as jnp
from jax import lax
from jax.experimental import pallas as pl
from jax.experimental.pallas import tpu as pltpu
from jax.experimental.pallas import tpu_sc as plsc

_B = 32768
_ND = 64
_L = 16
_ROWS = _B // _L
_TROWS = _ROWS // 16
_EXP_NEG5 = math.exp(-5.0)
_LN2 = 0.6931471805599453


def _softlog(x):
    bits = plsc.bitcast(x, jnp.int32)
    ex = (bits >> 23) & 0xFF
    m = plsc.bitcast((bits & 0x7FFFFF) | 0x3F800000, jnp.float32)
    big = m > 1.4142135381698608
    m2 = jnp.where(big, m * 0.5, m)
    ef = (ex - jnp.where(big, 126, 127)).astype(jnp.float32)
    r = (m2 - 1.0) / (m2 + 1.0)
    r2 = r * r
    poly = 1.0 + r2 * (1.0 / 3.0 + r2 * (1.0 / 5.0 + r2 * (1.0 / 7.0 + r2 * (1.0 / 9.0))))
    return ef * _LN2 + 2.0 * r * poly


def _body(s0_h, s1_h, lab_h, dat_h,
          out_h, parts_h, fin_h,
          s0_v, s1_v, lab_v, dat_v, p_v,
          acc, part_r, parts_v, t_tab, lz_tab, outv_r):
    sid = lax.axis_index("s")
    iota = lax.broadcasted_iota(jnp.int32, (_L,), 0)
    lane64 = iota * _ND

    r0 = sid * _TROWS
    pltpu.sync_copy(s0_h.at[pl.ds(r0, _TROWS)], s0_v)
    pltpu.sync_copy(s1_h.at[pl.ds(r0, _TROWS)], s1_v)
    pltpu.sync_copy(lab_h.at[pl.ds(r0, _TROWS)], lab_v)
    pltpu.sync_copy(dat_h.at[pl.ds(r0, _TROWS)], dat_v)

    def zero_body(j, c):
        acc[pl.ds(j * _L, _L)] = jnp.zeros((_L,), jnp.float32)
        return c
    lax.fori_loop(0, (32 * _ND) // _L, zero_body, 0)

    def p1_body(i, c):
        d = dat_v[i]
        idx = lane64 + ((d + iota) & (_ND - 1))
        x = s1_v[i] - s0_v[i]
        p = 1.0 / (1.0 + jnp.exp(-x))
        p_v[i] = p
        plsc.addupdate_scatter(acc, [idx], ((lab_v[i] << 12) + 1).astype(jnp.float32))
        plsc.addupdate_scatter(acc, [idx + 16 * _ND], jnp.exp(p))
        return c
    lax.fori_loop(0, _TROWS, p1_body, 0)
    plsc.subcore_barrier()

    for v in range(4):
        dv = v * _L + iota
        t_cn = jnp.zeros((_L,), jnp.float32)
        t_e = jnp.zeros((_L,), jnp.float32)
        for j in range(16):
            idx = j * _ND + ((dv + j) & (_ND - 1))
            t_cn = t_cn + plsc.load_gather(acc, [idx])
            t_e = t_e + plsc.load_gather(acc, [idx + 16 * _ND])
        icn = t_cn.astype(jnp.int32)
        part_r[pl.ds(v * _L, _L)] = (icn & 0xFFF).astype(jnp.float32)
        part_r[pl.ds(_ND + v * _L, _L)] = (icn >> 12).astype(jnp.float32)
        part_r[pl.ds(2 * _ND + v * _L, _L)] = t_e
    pltpu.sync_copy(part_r, parts_h.at[sid])
    plsc.subcore_barrier()
    pltpu.sync_copy(parts_h, parts_v)

    nvalid = jnp.float32(0.0)
    for v in range(4):
        cnt = parts_v[0, pl.ds(v * _L, _L)]
        n1 = parts_v[0, pl.ds(_ND + v * _L, _L)]
        e_tot = parts_v[0, pl.ds(2 * _ND + v * _L, _L)]
        for s in range(1, 16):
            cnt = cnt + parts_v[s, pl.ds(v * _L, _L)]
            n1 = n1 + parts_v[s, pl.ds(_ND + v * _L, _L)]
            e_tot = e_tot + parts_v[s, pl.ds(2 * _ND + v * _L, _L)]
        valid = cnt >= 2.0
        has1 = n1 > 0.5
        s_den = jnp.where(has1, n1 + (cnt - n1) * _EXP_NEG5, cnt)
        s_den = jnp.maximum(s_den, 1e-30)
        va = jnp.where(valid, jnp.where(has1, _EXP_NEG5, 1.0) / s_den, 0.0)
        vb = jnp.where(valid, 1.0 / s_den, 0.0)
        log_e = _softlog(jnp.maximum(e_tot, 1e-30))
        dv = v * _L + iota
        plsc.store_scatter(t_tab, [2 * dv], va)
        plsc.store_scatter(t_tab, [2 * dv + 1], vb)
        lz_tab[pl.ds(v * _L, _L)] = log_e
        nvalid = nvalid + jnp.sum(jnp.where(valid, 1.0, 0.0))

    def p4_body(i, acc_c):
        d = dat_v[i]
        t = plsc.load_gather(t_tab, [2 * d + lab_v[i]])
        log_e = plsc.load_gather(lz_tab, [d])
        return acc_c + t * (log_e - p_v[i])
    accv = lax.fori_loop(0, _TROWS, p4_body, jnp.zeros((_L,), jnp.float32))

    part = jnp.sum(accv)
    part_r[pl.ds(0, _L)] = jnp.where(iota == 0, part, 0.0)
    pltpu.sync_copy(part_r, fin_h.at[sid])
    plsc.subcore_barrier()

    @pl.when(sid == 0)
    def _():
        pltpu.sync_copy(fin_h, parts_v)
        tot = parts_v[0, pl.ds(0, _L)]
        for s in range(1, 16):
            tot = tot + parts_v[s, pl.ds(0, _L)]
        denv = jnp.maximum(nvalid, 1.0) + jnp.zeros((_L,), jnp.float32)
        outv_r[...] = jnp.where(iota == 0, tot, 0.0) / denv
        pltpu.sync_copy(outv_r, out_h)


_sc_loss = functools.partial(
    pl.kernel,
    out_type=(
        jax.ShapeDtypeStruct((_L,), jnp.float32),
        jax.ShapeDtypeStruct((16, 3 * _ND), jnp.float32),
        jax.ShapeDtypeStruct((16, 3 * _ND), jnp.float32),
    ),
    mesh=plsc.VectorSubcoreMesh(core_axis_name="c", subcore_axis_name="s",
                                num_cores=1),
    compiler_params=pltpu.CompilerParams(needs_layout_passes=False),
    scratch_types=[
        pltpu.VMEM((_TROWS, _L), jnp.float32),
        pltpu.VMEM((_TROWS, _L), jnp.float32),
        pltpu.VMEM((_TROWS, _L), jnp.int32),
        pltpu.VMEM((_TROWS, _L), jnp.int32),
        pltpu.VMEM((_TROWS, _L), jnp.float32),
        pltpu.VMEM((32 * _ND,), jnp.float32),
        pltpu.VMEM((3 * _ND,), jnp.float32),
        pltpu.VMEM((16, 3 * _ND), jnp.float32),
        pltpu.VMEM((2 * _ND,), jnp.float32),
        pltpu.VMEM((_ND,), jnp.float32),
        pltpu.VMEM((_L,), jnp.float32),
    ],
)(_body)


def kernel(scores, labels, dates):
    s0 = scores[:, 0].reshape(_ROWS, _L)
    s1 = scores[:, 1].reshape(_ROWS, _L)
    lab = labels.astype(jnp.int32).reshape(_ROWS, _L)
    dat = dates.astype(jnp.int32).reshape(_ROWS, _L)
    out, _, _ = _sc_loss(s0, s1, lab, dat)
    return out[0]


# speedup vs baseline: 1.5003x; 1.0308x over previous
import functools
import math

import jax
import jax.numpy as jnp
from jax import lax
from jax.experimental import pallas as pl
from jax.experimental.pallas import tpu as pltpu
from jax.experimental.pallas import tpu_sc as plsc

_B = 32768
_ND = 64
_L = 16
_ROWS = _B // _L
_TROWS = _ROWS // 16
_PARTW = 6 * _ND
_EXP_NEG5 = math.exp(-5.0)
_LN2 = 0.6931471805599453


def _softlog(x):
    bits = plsc.bitcast(x, jnp.int32)
    ex = (bits >> 23) & 0xFF
    m = plsc.bitcast((bits & 0x7FFFFF) | 0x3F800000, jnp.float32)
    big = m > 1.4142135381698608
    m2 = jnp.where(big, m * 0.5, m)
    ef = (ex - jnp.where(big, 126, 127)).astype(jnp.float32)
    r = (m2 - 1.0) / (m2 + 1.0)
    r2 = r * r
    poly = 1.0 + r2 * (1.0 / 3.0 + r2 * (1.0 / 5.0 + r2 * (1.0 / 7.0 + r2 * (1.0 / 9.0))))
    return ef * _LN2 + 2.0 * r * poly


def _body(s0_h, s1_h, lab_h, dat_h,
          out_h, parts_h,
          s0_v, s1_v, lab_v, dat_v,
          acc, part_r, parts_v, outv_r):
    sid = lax.axis_index("s")
    iota = lax.broadcasted_iota(jnp.int32, (_L,), 0)
    lane64 = iota * _ND

    r0 = sid * _TROWS
    pltpu.sync_copy(s0_h.at[pl.ds(r0, _TROWS)], s0_v)
    pltpu.sync_copy(s1_h.at[pl.ds(r0, _TROWS)], s1_v)
    pltpu.sync_copy(lab_h.at[pl.ds(r0, _TROWS)], lab_v)
    pltpu.sync_copy(dat_h.at[pl.ds(r0, _TROWS)], dat_v)

    def zero_body(j, c):
        acc[pl.ds(j * _L, _L)] = jnp.zeros((_L,), jnp.float32)
        return c
    lax.fori_loop(0, (64 * _ND) // _L, zero_body, 0)

    def p1_body(i, c):
        d = dat_v[i]
        l = lab_v[i]
        idx = lane64 + ((d + iota) & (_ND - 1))
        x = s1_v[i] - s0_v[i]
        p = 1.0 / (1.0 + jnp.exp(-x))
        plsc.addupdate_scatter(acc, [idx], ((l << 12) + 1).astype(jnp.float32))
        plsc.addupdate_scatter(acc, [idx + 16 * _ND], jnp.exp(p))
        plsc.addupdate_scatter(acc, [idx + 32 * _ND + (l << 10)], p)
        return c
    lax.fori_loop(0, _TROWS, p1_body, 0)
    plsc.subcore_barrier()

    for v in range(4):
        dv = v * _L + iota
        t_cn = jnp.zeros((_L,), jnp.float32)
        t_e = jnp.zeros((_L,), jnp.float32)
        t_p0 = jnp.zeros((_L,), jnp.float32)
        t_p1 = jnp.zeros((_L,), jnp.float32)
        for j in range(16):
            idx = j * _ND + ((dv + j) & (_ND - 1))
            t_cn = t_cn + plsc.load_gather(acc, [idx])
            t_e = t_e + plsc.load_gather(acc, [idx + 16 * _ND])
            t_p0 = t_p0 + plsc.load_gather(acc, [idx + 32 * _ND])
            t_p1 = t_p1 + plsc.load_gather(acc, [idx + 48 * _ND])
        icn = t_cn.astype(jnp.int32)
        part_r[pl.ds(v * _L, _L)] = (icn & 0xFFF).astype(jnp.float32)
        part_r[pl.ds(_ND + v * _L, _L)] = (icn >> 12).astype(jnp.float32)
        part_r[pl.ds(2 * _ND + v * _L, _L)] = t_e
        part_r[pl.ds(3 * _ND + v * _L, _L)] = t_p0
        part_r[pl.ds(4 * _ND + v * _L, _L)] = t_p1
    pltpu.sync_copy(part_r, parts_h.at[sid])
    plsc.subcore_barrier()

    @pl.when(sid == 0)
    def _():
        pltpu.sync_copy(parts_h, parts_v)
        nvalid = jnp.float32(0.0)
        acc_ce = jnp.zeros((_L,), jnp.float32)
        for v in range(4):
            cnt = parts_v[0, pl.ds(v * _L, _L)]
            n1 = parts_v[0, pl.ds(_ND + v * _L, _L)]
            e_tot = parts_v[0, pl.ds(2 * _ND + v * _L, _L)]
            p0 = parts_v[0, pl.ds(3 * _ND + v * _L, _L)]
            p1 = parts_v[0, pl.ds(4 * _ND + v * _L, _L)]
            for s in range(1, 16):
                cnt = cnt + parts_v[s, pl.ds(v * _L, _L)]
                n1 = n1 + parts_v[s, pl.ds(_ND + v * _L, _L)]
                e_tot = e_tot + parts_v[s, pl.ds(2 * _ND + v * _L, _L)]
                p0 = p0 + parts_v[s, pl.ds(3 * _ND + v * _L, _L)]
                p1 = p1 + parts_v[s, pl.ds(4 * _ND + v * _L, _L)]
            valid = cnt >= 2.0
            has1 = n1 > 0.5
            s_den = jnp.where(has1, n1 + (cnt - n1) * _EXP_NEG5, cnt)
            s_den = jnp.maximum(s_den, 1e-30)
            coef_a = jnp.where(has1, _EXP_NEG5, 1.0) / s_den
            coef_b = 1.0 / s_den
            log_e = _softlog(jnp.maximum(e_tot, 1e-30))
            ce = log_e - coef_a * p0 - coef_b * p1
            acc_ce = acc_ce + jnp.where(valid, ce, 0.0)
            nvalid = nvalid + jnp.sum(jnp.where(valid, 1.0, 0.0))
        num = jnp.sum(acc_ce)
        denv = jnp.maximum(nvalid, 1.0) + jnp.zeros((_L,), jnp.float32)
        outv_r[...] = jnp.where(iota == 0, num, 0.0) / denv
        pltpu.sync_copy(outv_r, out_h)


_sc_loss = functools.partial(
    pl.kernel,
    out_type=(
        jax.ShapeDtypeStruct((_L,), jnp.float32),
        jax.ShapeDtypeStruct((16, _PARTW), jnp.float32),
    ),
    mesh=plsc.VectorSubcoreMesh(core_axis_name="c", subcore_axis_name="s",
                                num_cores=1),
    compiler_params=pltpu.CompilerParams(needs_layout_passes=False),
    scratch_types=[
        pltpu.VMEM((_TROWS, _L), jnp.float32),
        pltpu.VMEM((_TROWS, _L), jnp.float32),
        pltpu.VMEM((_TROWS, _L), jnp.int32),
        pltpu.VMEM((_TROWS, _L), jnp.int32),
        pltpu.VMEM((64 * _ND,), jnp.float32),
        pltpu.VMEM((_PARTW,), jnp.float32),
        pltpu.VMEM((16, _PARTW), jnp.float32),
        pltpu.VMEM((_L,), jnp.float32),
    ],
)(_body)


def kernel(scores, labels, dates):
    s0 = scores[:, 0].reshape(_ROWS, _L)
    s1 = scores[:, 1].reshape(_ROWS, _L)
    lab = labels.astype(jnp.int32).reshape(_ROWS, _L)
    dat = dates.astype(jnp.int32).reshape(_ROWS, _L)
    out, _ = _sc_loss(s0, s1, lab, dat)
    return out[0]
